# Initial kernel scaffold; baseline (speedup 1.0000x reference)
import jax
import jax.numpy as jnp
from jax.experimental import pallas as pl


def kernel(x, W):
    raise NotImplementedError("write your pallas kernel here")


# speedup vs baseline: 153.6515x; 153.6515x over previous
import jax
import jax.numpy as jnp
from jax import lax
from jax.experimental import pallas as pl
from jax.experimental.pallas import tpu as pltpu, tpu_sc as plsc

B, L, C, V = 1024, 8192, 4, 16
NC, NS = 2, 16
NW = NC * NS
ROWS_PER_W = B // NW
LANES = 16
UNROLL = 8


def _sc_body(x_hbm, w_hbm, out_hbm, w_v, x_v, out_v):
    cid = lax.axis_index("c")
    sid = lax.axis_index("s")
    wid = sid * NC + cid

    pltpu.sync_copy(w_hbm, w_v)
    wcols = [w_v[c] for c in range(C)]
    dnums = lax.GatherDimensionNumbers(
        offset_dims=(), collapsed_slice_dims=(0,), start_index_map=(0,))

    def row_body(r, carry):
        row = wid * ROWS_PER_W + r
        pltpu.sync_copy(x_hbm.at[row], x_v)

        def chunk_body(j, carry2):
            for u in range(UNROLL):
                base = (j * UNROLL + u) * LANES
                v = x_v[pl.ds(base, LANES)]
                vi = v.reshape(LANES, 1)
                for c in range(C):
                    out_v[c, pl.ds(base, LANES)] = lax.gather(
                        wcols[c], vi, dnums, slice_sizes=(1,),
                        mode=lax.GatherScatterMode.PROMISE_IN_BOUNDS)
            return carry2

        lax.fori_loop(0, L // (LANES * UNROLL), chunk_body, 0)
        pltpu.sync_copy(out_v, out_hbm.at[row])
        return carry

    lax.fori_loop(0, ROWS_PER_W, row_body, 0)


@jax.jit
def kernel(x, W):
    mesh = plsc.VectorSubcoreMesh(core_axis_name="c", subcore_axis_name="s")
    k = pl.kernel(
        _sc_body,
        out_type=jax.ShapeDtypeStruct((B, C, L), jnp.float32),
        mesh=mesh,
        scratch_types=[
            pltpu.VMEM((C, V), jnp.float32),
            pltpu.VMEM((L,), jnp.int32),
            pltpu.VMEM((C, L), jnp.float32),
        ],
    )
    return k(x.astype(jnp.int32), W.T.reshape(C, V))


# speedup vs baseline: 393.9896x; 2.5642x over previous
import jax
import jax.numpy as jnp
from jax import lax
from jax.experimental import pallas as pl
from jax.experimental.pallas import tpu as pltpu, tpu_sc as plsc

B, L, C, V = 1024, 8192, 4, 16
NC, NS = 2, 16
NW = NC * NS
ROWS_PER_W = B // NW
LANES = 16
UNROLL = 16


def _sc_body(x_hbm, w_hbm, out_hbm, w_v, x0, x1, o0, o1, sem_in, sem_out):
    cid = lax.axis_index("c")
    sid = lax.axis_index("s")
    wid = sid * NC + cid
    base_row = wid * ROWS_PER_W

    dnums = lax.GatherDimensionNumbers(
        offset_dims=(), collapsed_slice_dims=(0,), start_index_map=(0,))

    def vgather(data, idx):
        return lax.gather(data, idx.reshape(LANES, 1), dnums,
                          slice_sizes=(1,),
                          mode=lax.GatherScatterMode.PROMISE_IN_BOUNDS)

    pltpu.sync_copy(w_hbm, w_v)
    wcols = [w_v[c] for c in range(C)]

    xbufs = (x0, x1)
    obufs = (o0, o1)

    def compute_row(xb, ob):
        def chunk_body(j, carry):
            offs = [(j * UNROLL + u) * LANES for u in range(UNROLL)]
            vis = [xb[pl.ds(off, LANES)] for off in offs]
            for u, off in enumerate(offs):
                for c in range(C):
                    ob[c, pl.ds(off, LANES)] = vgather(wcols[c], vis[u])
            return carry
        lax.fori_loop(0, L // (LANES * UNROLL), chunk_body, 0)

    pltpu.async_copy(x_hbm.at[pl.ds(base_row, 2)], x0, sem_in)

    def super_body(s, carry):
        for q in range(2):
            g = s * 2 + q
            xb = xbufs[q]
            pltpu.make_async_copy(
                x_hbm.at[pl.ds(base_row + g * 2, 2)], xb, sem_in).wait()
            @pl.when(g < ROWS_PER_W // 2 - 1)
            def _():
                pltpu.async_copy(
                    x_hbm.at[pl.ds(base_row + g * 2 + 2, 2)],
                    xbufs[1 - q], sem_in)
            for p in range(2):
                r = g * 2 + p
                row = base_row + r
                ob = obufs[p]
                @pl.when(r >= 2)
                def _():
                    pltpu.make_async_copy(
                        ob, out_hbm.at[row - 2], sem_out).wait()
                compute_row(xb.at[p], ob)
                pltpu.async_copy(ob, out_hbm.at[row], sem_out)
        return carry

    lax.fori_loop(0, ROWS_PER_W // 4, super_body, 0)
    last = base_row + ROWS_PER_W
    pltpu.make_async_copy(o0, out_hbm.at[last - 2], sem_out).wait()
    pltpu.make_async_copy(o1, out_hbm.at[last - 1], sem_out).wait()


@jax.jit
def kernel(x, W):
    mesh = plsc.VectorSubcoreMesh(core_axis_name="c", subcore_axis_name="s")
    k = pl.kernel(
        _sc_body,
        out_type=jax.ShapeDtypeStruct((B, C, L), jnp.float32),
        mesh=mesh,
        scratch_types=[
            pltpu.VMEM((C, V), jnp.float32),
            pltpu.VMEM((2, L), jnp.int32),
            pltpu.VMEM((2, L), jnp.int32),
            pltpu.VMEM((C, L), jnp.float32),
            pltpu.VMEM((C, L), jnp.float32),
            pltpu.SemaphoreType.DMA,
            pltpu.SemaphoreType.DMA,
        ],
    )
    return k(x.astype(jnp.int32), W.T.reshape(C, V))
